# baseline (speedup 1.0000x reference)
import functools

import jax
import jax.numpy as jnp
from jax import lax
from jax.experimental import pallas as pl
from jax.experimental.pallas import tpu as pltpu
from jax.experimental.pallas import tpu_sc as plsc

B = 16384
D = 64
V = 5
_L = 16

_info = plsc.get_sparse_core_info()
_NC, _NS = _info.num_cores, _info.num_subcores
_NW = _NC * _NS
_BPW = B // _NW
_NG = _BPW // _L

_mesh = plsc.VectorSubcoreMesh(core_axis_name="c", subcore_axis_name="s")


@functools.partial(
    pl.kernel,
    mesh=_mesh,
    out_type=jax.ShapeDtypeStruct((D, B), jnp.float32),
    scratch_types=[
        pltpu.VMEM((V, D), jnp.float32),
        pltpu.VMEM((_BPW,), jnp.int32),
        pltpu.VMEM((D, _BPW), jnp.float32),
    ],
    compiler_params=pltpu.CompilerParams(use_tc_tiling_on_sc=True,
                                         needs_layout_passes=False),
)
def _gather_kernel(table_hbm, idx_hbm, out_hbm, table_v, idx_v, tbuf):
    cid = lax.axis_index("c")
    sid = lax.axis_index("s")
    wid = sid * _NC + cid
    base = wid * _BPW

    pltpu.sync_copy(table_hbm, table_v)
    pltpu.sync_copy(idx_hbm.at[wid], idx_v)

    def group(g, carry):
        idxs = idx_v[pl.ds(g * _L, _L)]
        for d in range(D):
            dvec = jnp.full((_L,), d, jnp.int32)
            tbuf[d, pl.ds(g * _L, _L)] = plsc.load_gather(table_v,
                                                          [idxs, dvec])
        return carry

    lax.fori_loop(0, _NG, group, 0)

    pltpu.sync_copy(tbuf, out_hbm.at[:, pl.ds(base, _BPW)])


def kernel(command, embed_table):
    idx = command.reshape(_NW, _BPW).astype(jnp.int32)
    return _gather_kernel(embed_table, idx).T


# speedup vs baseline: 1.5072x; 1.5072x over previous
import functools

import jax
import jax.numpy as jnp
from jax import lax
from jax.experimental import pallas as pl
from jax.experimental.pallas import tpu as pltpu
from jax.experimental.pallas import tpu_sc as plsc

B = 16384
D = 64
V = 5

_info = plsc.get_sparse_core_info()
_NC, _NS = _info.num_cores, _info.num_subcores
_NW = _NC * _NS
_BPW = B // _NW
_C = 128
_NCH = _BPW // _C

_mesh = plsc.VectorSubcoreMesh(core_axis_name="c", subcore_axis_name="s")


@functools.partial(
    pl.kernel,
    mesh=_mesh,
    out_type=jax.ShapeDtypeStruct((B, 128), jnp.float32),
    scratch_types=[
        pltpu.VMEM_SHARED((V, 128), jnp.float32),
        pltpu.VMEM((_NCH, _C), jnp.int32),
        pltpu.VMEM((_BPW, 128), jnp.float32),
        pltpu.SemaphoreType.DMA,
        pltpu.SemaphoreType.DMA,
    ],
    compiler_params=pltpu.CompilerParams(use_tc_tiling_on_sc=False),
)
def _gather_kernel(table_hbm, idx_hbm, out_hbm, table_sh, idx_v, rows_v,
                   gsem, wsem):
    cid = lax.axis_index("c")
    sid = lax.axis_index("s")
    wid = sid * _NC + cid
    base = wid * _BPW

    @pl.when(sid == 0)
    def _stage_table():
        pltpu.sync_copy(table_hbm, table_sh)

    plsc.subcore_barrier()

    pltpu.sync_copy(idx_hbm.at[wid], idx_v)

    gathers = [
        pltpu.async_copy(table_sh.at[idx_v.at[k]],
                         rows_v.at[pl.ds(k * _C, _C)], gsem)
        for k in range(_NCH)
    ]
    writes = []
    for k in range(_NCH):
        gathers[k].wait()
        writes.append(
            pltpu.async_copy(rows_v.at[pl.ds(k * _C, _C)],
                             out_hbm.at[pl.ds(base + k * _C, _C)], wsem))
    for w in writes:
        w.wait()


def kernel(command, embed_table):
    idx = command.reshape(_NW, _NCH, _C).astype(jnp.int32)
    table_p = jnp.zeros((V, 128), jnp.float32).at[:, :D].set(embed_table)
    return _gather_kernel(table_p, idx)[:, :D]
